# baseline (speedup 1.0000x reference)
import functools

import jax
import jax.numpy as jnp
from jax import lax
from jax.experimental import pallas as pl
from jax.experimental.pallas import tpu as pltpu
from jax.experimental.pallas import tpu_sc as plsc

D = 128
N_ITEM = 10000
N_USER = 10000
E = 320000

NS = 16
BLK = 128
NB = 160
E_PAD = NS * NB * BLK
ACC_ROWS = 10240
ROWS_PER_TILE = ACC_ROWS // NS


def _sc_aggregate(table, src_idx, dst_idx, zrows, ones_rows):
    mesh = plsc.VectorSubcoreMesh(core_axis_name="c", subcore_axis_name="s")

    @functools.partial(
        pl.kernel,
        out_type=(
            jax.ShapeDtypeStruct((2, ACC_ROWS, D), jnp.float32),
            jax.ShapeDtypeStruct((2, ACC_ROWS, D), jnp.float32),
        ),
        mesh=mesh,
        scratch_types=[
            pltpu.VMEM((BLK,), jnp.int32),
            pltpu.VMEM((BLK,), jnp.int32),
            pltpu.VMEM((BLK,), jnp.int32),
            pltpu.VMEM((BLK,), jnp.int32),
            pltpu.VMEM((BLK, D), jnp.float32),
            pltpu.VMEM((BLK, D), jnp.float32),
            pltpu.VMEM((16, D), jnp.float32),
            pltpu.VMEM_SHARED((ACC_ROWS, D), jnp.float32),
            pltpu.SemaphoreType.DMA,
            pltpu.SemaphoreType.DMA,
            pltpu.SemaphoreType.DMA,
            pltpu.SemaphoreType.DMA,
            pltpu.SemaphoreType.DMA,
            pltpu.SemaphoreType.DMA,
        ],
    )
    def k(table_hbm, src_hbm, dst_hbm, z_hbm, ones_hbm, feat_out, deg_out,
          src_a, src_b, dst_a, dst_b, rows_a, rows_b, zf_v, acc_f,
          isem_a, isem_b, gsem_a, gsem_b, ssem_a, ssem_b):
        cid = lax.axis_index("c")
        sid = lax.axis_index("s")

        pltpu.sync_copy(z_hbm, zf_v)

        def zero_acc():
            def zbody(t, carry):
                r = sid * ROWS_PER_TILE + t * 16
                pltpu.sync_copy(zf_v, acc_f.at[pl.ds(r, 16)])
                return carry

            lax.fori_loop(0, ROWS_PER_TILE // 16, zbody, 0)

        def dump(out_ref):
            r0 = sid * ROWS_PER_TILE
            pltpu.sync_copy(acc_f.at[pl.ds(r0, ROWS_PER_TILE)],
                            out_ref.at[cid, pl.ds(r0, ROWS_PER_TILE)])

        zero_acc()
        plsc.subcore_barrier()

        def fetch_idx(b, sv, dv, sem):
            s1 = pltpu.async_copy(src_hbm.at[cid, sid, b], sv, sem)
            s2 = pltpu.async_copy(dst_hbm.at[cid, sid, b], dv, sem)
            return s1, s2

        def wait_idx(sv, dv, sem):
            pltpu.make_async_copy(src_hbm.at[cid, sid, 0], sv, sem).wait()
            pltpu.make_async_copy(dst_hbm.at[cid, sid, 0], dv, sem).wait()

        pltpu.sync_copy(src_hbm.at[cid, sid, 0], src_a)
        pltpu.sync_copy(dst_hbm.at[cid, sid, 0], dst_a)
        fetch_idx(1, src_b, dst_b, isem_b)
        pltpu.async_copy(table_hbm.at[src_a], rows_a, gsem_a)

        def body(t, carry):
            b0 = 2 * t
            pltpu.make_async_copy(table_hbm.at[src_a], rows_a, gsem_a).wait()
            wait_idx(src_b, dst_b, isem_b)
            pltpu.async_copy(table_hbm.at[src_b], rows_b, gsem_b)
            pltpu.sync_copy(rows_a, acc_f.at[dst_a], add=True)

            @pl.when(t < NB // 2 - 1)
            def _():
                fetch_idx(b0 + 2, src_a, dst_a, isem_a)
                wait_idx(src_a, dst_a, isem_a)
                pltpu.async_copy(table_hbm.at[src_a], rows_a, gsem_a)

            pltpu.make_async_copy(table_hbm.at[src_b], rows_b, gsem_b).wait()
            pltpu.sync_copy(rows_b, acc_f.at[dst_b], add=True)

            @pl.when(t < NB // 2 - 1)
            def _():
                fetch_idx(b0 + 3, src_b, dst_b, isem_b)

            return carry

        lax.fori_loop(0, NB // 2, body, 0)
        plsc.subcore_barrier()
        dump(feat_out)
        plsc.subcore_barrier()

        zero_acc()
        pltpu.sync_copy(ones_hbm, rows_a)
        plsc.subcore_barrier()

        pltpu.sync_copy(dst_hbm.at[cid, sid, 0], dst_a)
        pltpu.async_copy(dst_hbm.at[cid, sid, 1], dst_b, isem_b)

        def dbody(t, carry):
            b0 = 2 * t
            s_a = pltpu.async_copy(rows_a, acc_f.at[dst_a], ssem_a, add=True)
            pltpu.make_async_copy(dst_hbm.at[cid, sid, 0], dst_b, isem_b).wait()
            s_b = pltpu.async_copy(rows_a, acc_f.at[dst_b], ssem_b, add=True)
            s_a.wait()

            @pl.when(t < NB // 2 - 1)
            def _():
                pltpu.async_copy(dst_hbm.at[cid, sid, b0 + 2], dst_a, isem_a)
                pltpu.make_async_copy(dst_hbm.at[cid, sid, 0], dst_a,
                                      isem_a).wait()

            s_b.wait()

            @pl.when(t < NB // 2 - 1)
            def _():
                pltpu.async_copy(dst_hbm.at[cid, sid, b0 + 3], dst_b, isem_b)

            return carry

        lax.fori_loop(0, NB // 2, dbody, 0)
        plsc.subcore_barrier()
        dump(deg_out)

    return k(table, src_idx, dst_idx, zrows, ones_rows)


def _tc_epilogue_body(feat, dall, wc, ws, out):
    deg0 = jnp.maximum(dall[0, :, 0:1], 1.0)
    deg1 = jnp.maximum(dall[1, :, 0:1], 1.0)
    a0 = feat[0] / deg0
    a1 = feat[1] / deg1
    dn = (((1,), (0,)), ((), ()))
    p = lax.dot_general(a0, wc[...], dn, precision=lax.Precision.HIGHEST,
                        preferred_element_type=jnp.float32)
    q = lax.dot_general(a1, ws[...], dn, precision=lax.Precision.HIGHEST,
                        preferred_element_type=jnp.float32)
    out[...] = 0.5 * (p + q)


def _tc_epilogue(feat, deg, W_clicks, W_similar):
    BR = 2048
    grid = (ACC_ROWS // BR,)
    out = pl.pallas_call(
        _tc_epilogue_body,
        grid=grid,
        in_specs=[
            pl.BlockSpec((2, BR, D), lambda i: (0, i, 0)),
            pl.BlockSpec((2, BR, D), lambda i: (0, i, 0)),
            pl.BlockSpec((D, D), lambda i: (0, 0)),
            pl.BlockSpec((D, D), lambda i: (0, 0)),
        ],
        out_specs=pl.BlockSpec((BR, D), lambda i: (i, 0)),
        out_shape=jax.ShapeDtypeStruct((ACC_ROWS, D), jnp.float32),
    )(feat, deg, W_clicks, W_similar)
    return out[:N_ITEM]


def kernel(x_user, x_item, edge_index_clicks, edge_index_similar,
           W_clicks, W_similar):
    table = jnp.concatenate([x_user, x_item], axis=0)

    src_c = edge_index_clicks[0].astype(jnp.int32)
    dst_c = edge_index_clicks[1].astype(jnp.int32)
    src_s = edge_index_similar[0].astype(jnp.int32) + N_USER
    dst_s = edge_index_similar[1].astype(jnp.int32)

    pad = E_PAD - E
    src_c = jnp.concatenate([src_c, jnp.zeros((pad,), jnp.int32)])
    src_s = jnp.concatenate([src_s, jnp.full((pad,), N_USER, jnp.int32)])
    dpad = jnp.full((pad,), N_ITEM, jnp.int32)
    dst_c = jnp.concatenate([dst_c, dpad])
    dst_s = jnp.concatenate([dst_s, dpad])

    src_idx = jnp.stack([src_c, src_s]).reshape(2, NS, NB, BLK)
    dst_idx = jnp.stack([dst_c, dst_s]).reshape(2, NS, NB, BLK)

    zrows = jnp.zeros((16, D), jnp.float32)
    ones_rows = jnp.ones((BLK, D), jnp.float32)

    feat, deg = _sc_aggregate(table, src_idx, dst_idx, zrows, ones_rows)
    return _tc_epilogue(feat, deg, W_clicks, W_similar)


# speedup vs baseline: 1.0743x; 1.0743x over previous
import functools

import jax
import jax.numpy as jnp
from jax import lax
from jax.experimental import pallas as pl
from jax.experimental.pallas import tpu as pltpu
from jax.experimental.pallas import tpu_sc as plsc

D = 128
N_ITEM = 10000
N_USER = 10000
E = 320000

NS = 16
BLK = 160
NB = 128
E_PAD = NS * NB * BLK
ACC_ROWS = 10240
ROWS_PER_TILE = ACC_ROWS // NS


def _sc_aggregate(table, src_idx, dst_idx, zrows, ones_rows):
    mesh = plsc.VectorSubcoreMesh(core_axis_name="c", subcore_axis_name="s")

    @functools.partial(
        pl.kernel,
        out_type=(
            jax.ShapeDtypeStruct((2, ACC_ROWS, D), jnp.float32),
            jax.ShapeDtypeStruct((2, ACC_ROWS, D), jnp.float32),
        ),
        mesh=mesh,
        scratch_types=[
            pltpu.VMEM((BLK,), jnp.int32),
            pltpu.VMEM((BLK,), jnp.int32),
            pltpu.VMEM((BLK,), jnp.int32),
            pltpu.VMEM((BLK,), jnp.int32),
            pltpu.VMEM((BLK, D), jnp.float32),
            pltpu.VMEM((BLK, D), jnp.float32),
            pltpu.VMEM((16, D), jnp.float32),
            pltpu.VMEM_SHARED((ACC_ROWS, D), jnp.float32),
            pltpu.SemaphoreType.DMA,
            pltpu.SemaphoreType.DMA,
            pltpu.SemaphoreType.DMA,
            pltpu.SemaphoreType.DMA,
            pltpu.SemaphoreType.DMA,
            pltpu.SemaphoreType.DMA,
        ],
    )
    def k(table_hbm, src_hbm, dst_hbm, z_hbm, ones_hbm, feat_out, deg_out,
          src_a, src_b, dst_a, dst_b, rows_a, rows_b, zf_v, acc_f,
          isem_a, isem_b, gsem_a, gsem_b, ssem_a, ssem_b):
        cid = lax.axis_index("c")
        sid = lax.axis_index("s")

        pltpu.sync_copy(z_hbm, zf_v)

        def zero_acc():
            def zbody(t, carry):
                r = sid * ROWS_PER_TILE + t * 16
                pltpu.sync_copy(zf_v, acc_f.at[pl.ds(r, 16)])
                return carry

            lax.fori_loop(0, ROWS_PER_TILE // 16, zbody, 0)

        def dump(out_ref):
            r0 = sid * ROWS_PER_TILE
            pltpu.sync_copy(acc_f.at[pl.ds(r0, ROWS_PER_TILE)],
                            out_ref.at[cid, pl.ds(r0, ROWS_PER_TILE)])

        zero_acc()
        plsc.subcore_barrier()

        def fetch_idx(b, sv, dv, sem):
            s1 = pltpu.async_copy(src_hbm.at[cid, sid, b], sv, sem)
            s2 = pltpu.async_copy(dst_hbm.at[cid, sid, b], dv, sem)
            return s1, s2

        def wait_idx(sv, dv, sem):
            pltpu.make_async_copy(src_hbm.at[cid, sid, 0], sv, sem).wait()
            pltpu.make_async_copy(dst_hbm.at[cid, sid, 0], dv, sem).wait()

        pltpu.sync_copy(src_hbm.at[cid, sid, 0], src_a)
        pltpu.sync_copy(dst_hbm.at[cid, sid, 0], dst_a)
        fetch_idx(1, src_b, dst_b, isem_b)
        pltpu.async_copy(table_hbm.at[src_a], rows_a, gsem_a)

        def body(t, carry):
            b0 = 2 * t
            pltpu.make_async_copy(table_hbm.at[src_a], rows_a, gsem_a).wait()
            wait_idx(src_b, dst_b, isem_b)
            pltpu.async_copy(table_hbm.at[src_b], rows_b, gsem_b)
            pltpu.sync_copy(rows_a, acc_f.at[dst_a], add=True)

            @pl.when(t < NB // 2 - 1)
            def _():
                fetch_idx(b0 + 2, src_a, dst_a, isem_a)
                wait_idx(src_a, dst_a, isem_a)
                pltpu.async_copy(table_hbm.at[src_a], rows_a, gsem_a)

            pltpu.make_async_copy(table_hbm.at[src_b], rows_b, gsem_b).wait()
            pltpu.sync_copy(rows_b, acc_f.at[dst_b], add=True)

            @pl.when(t < NB // 2 - 1)
            def _():
                fetch_idx(b0 + 3, src_b, dst_b, isem_b)

            return carry

        lax.fori_loop(0, NB // 2, body, 0)
        plsc.subcore_barrier()
        dump(feat_out)
        plsc.subcore_barrier()

        zero_acc()
        pltpu.sync_copy(ones_hbm, rows_a)
        plsc.subcore_barrier()

        pltpu.sync_copy(dst_hbm.at[cid, sid, 0], dst_a)
        pltpu.async_copy(dst_hbm.at[cid, sid, 1], dst_b, isem_b)

        def dbody(t, carry):
            b0 = 2 * t
            s_a = pltpu.async_copy(rows_a, acc_f.at[dst_a], ssem_a, add=True)
            pltpu.make_async_copy(dst_hbm.at[cid, sid, 0], dst_b, isem_b).wait()
            s_b = pltpu.async_copy(rows_a, acc_f.at[dst_b], ssem_b, add=True)
            s_a.wait()

            @pl.when(t < NB // 2 - 1)
            def _():
                pltpu.async_copy(dst_hbm.at[cid, sid, b0 + 2], dst_a, isem_a)
                pltpu.make_async_copy(dst_hbm.at[cid, sid, 0], dst_a,
                                      isem_a).wait()

            s_b.wait()

            @pl.when(t < NB // 2 - 1)
            def _():
                pltpu.async_copy(dst_hbm.at[cid, sid, b0 + 3], dst_b, isem_b)

            return carry

        lax.fori_loop(0, NB // 2, dbody, 0)
        plsc.subcore_barrier()
        dump(deg_out)

    return k(table, src_idx, dst_idx, zrows, ones_rows)


def _tc_epilogue_body(feat, dall, wc, ws, out):
    deg0 = jnp.maximum(dall[0, :, 0:1], 1.0)
    deg1 = jnp.maximum(dall[1, :, 0:1], 1.0)
    a0 = feat[0] / deg0
    a1 = feat[1] / deg1
    dn = (((1,), (0,)), ((), ()))
    p = lax.dot_general(a0, wc[...], dn, precision=lax.Precision.HIGHEST,
                        preferred_element_type=jnp.float32)
    q = lax.dot_general(a1, ws[...], dn, precision=lax.Precision.HIGHEST,
                        preferred_element_type=jnp.float32)
    out[...] = 0.5 * (p + q)


def _tc_epilogue(feat, deg, W_clicks, W_similar):
    BR = 2048
    grid = (ACC_ROWS // BR,)
    out = pl.pallas_call(
        _tc_epilogue_body,
        grid=grid,
        in_specs=[
            pl.BlockSpec((2, BR, D), lambda i: (0, i, 0)),
            pl.BlockSpec((2, BR, D), lambda i: (0, i, 0)),
            pl.BlockSpec((D, D), lambda i: (0, 0)),
            pl.BlockSpec((D, D), lambda i: (0, 0)),
        ],
        out_specs=pl.BlockSpec((BR, D), lambda i: (i, 0)),
        out_shape=jax.ShapeDtypeStruct((ACC_ROWS, D), jnp.float32),
    )(feat, deg, W_clicks, W_similar)
    return out[:N_ITEM]


def kernel(x_user, x_item, edge_index_clicks, edge_index_similar,
           W_clicks, W_similar):
    table = jnp.concatenate([x_user, x_item], axis=0)

    src_c = edge_index_clicks[0].astype(jnp.int32)
    dst_c = edge_index_clicks[1].astype(jnp.int32)
    src_s = edge_index_similar[0].astype(jnp.int32) + N_USER
    dst_s = edge_index_similar[1].astype(jnp.int32)

    pad = E_PAD - E
    src_c = jnp.concatenate([src_c, jnp.zeros((pad,), jnp.int32)])
    src_s = jnp.concatenate([src_s, jnp.full((pad,), N_USER, jnp.int32)])
    dpad = jnp.full((pad,), N_ITEM, jnp.int32)
    dst_c = jnp.concatenate([dst_c, dpad])
    dst_s = jnp.concatenate([dst_s, dpad])

    src_idx = jnp.stack([src_c, src_s]).reshape(2, NS, NB, BLK)
    dst_idx = jnp.stack([dst_c, dst_s]).reshape(2, NS, NB, BLK)

    zrows = jnp.zeros((16, D), jnp.float32)
    ones_rows = jnp.ones((BLK, D), jnp.float32)

    feat, deg = _sc_aggregate(table, src_idx, dst_idx, zrows, ones_rows)
    return _tc_epilogue(feat, deg, W_clicks, W_similar)


# speedup vs baseline: 1.8646x; 1.7357x over previous
import functools

import jax
import jax.numpy as jnp
from jax import lax
from jax.experimental import pallas as pl
from jax.experimental.pallas import tpu as pltpu
from jax.experimental.pallas import tpu_sc as plsc

D = 128
N_ITEM = 10000
N_USER = 10000
E = 320000

NS = 16
BLK = 176
NB = 114
E_PAD = NS * NB * BLK
ACC_ROWS = 10112
ROWS_PER_TILE = ACC_ROWS // NS


def _sc_aggregate(table, src_idx, dst_idx, zrows, ones_rows):
    mesh = plsc.VectorSubcoreMesh(core_axis_name="c", subcore_axis_name="s")

    @functools.partial(
        pl.kernel,
        out_type=(
            jax.ShapeDtypeStruct((2, ACC_ROWS, D), jnp.float32),
            jax.ShapeDtypeStruct((2, ACC_ROWS, D), jnp.float32),
        ),
        mesh=mesh,
        scratch_types=[
            pltpu.VMEM((BLK,), jnp.int32),
            pltpu.VMEM((BLK,), jnp.int32),
            pltpu.VMEM((BLK,), jnp.int32),
            pltpu.VMEM((BLK,), jnp.int32),
            pltpu.VMEM((BLK, D), jnp.float32),
            pltpu.VMEM((BLK, D), jnp.float32),
            pltpu.VMEM((16, D), jnp.float32),
            pltpu.VMEM_SHARED((ACC_ROWS, D), jnp.float32),
            pltpu.SemaphoreType.DMA,
            pltpu.SemaphoreType.DMA,
            pltpu.SemaphoreType.DMA,
            pltpu.SemaphoreType.DMA,
            pltpu.SemaphoreType.DMA,
            pltpu.SemaphoreType.DMA,
        ],
    )
    def k(table_hbm, src_hbm, dst_hbm, z_hbm, ones_hbm, feat_out, deg_out,
          src_a, src_b, dst_a, dst_b, rows_a, rows_b, zf_v, acc_f,
          isem_a, isem_b, gsem_a, gsem_b, ssem_a, ssem_b):
        cid = lax.axis_index("c")
        sid = lax.axis_index("s")

        pltpu.sync_copy(z_hbm, zf_v)

        def zero_acc():
            def zbody(t, carry):
                r = sid * ROWS_PER_TILE + t * 16
                pltpu.sync_copy(zf_v, acc_f.at[pl.ds(r, 16)])
                return carry

            lax.fori_loop(0, ROWS_PER_TILE // 16, zbody, 0)
            rem = ROWS_PER_TILE % 16
            if rem:
                r = sid * ROWS_PER_TILE + (ROWS_PER_TILE // 16) * 16
                pltpu.sync_copy(zf_v.at[pl.ds(0, rem)], acc_f.at[pl.ds(r, rem)])

        def dump(out_ref):
            r0 = sid * ROWS_PER_TILE
            pltpu.sync_copy(acc_f.at[pl.ds(r0, ROWS_PER_TILE)],
                            out_ref.at[cid, pl.ds(r0, ROWS_PER_TILE)])

        zero_acc()
        plsc.subcore_barrier()

        def fetch_idx(b, sv, dv, sem):
            s1 = pltpu.async_copy(src_hbm.at[cid, sid, b], sv, sem)
            s2 = pltpu.async_copy(dst_hbm.at[cid, sid, b], dv, sem)
            return s1, s2

        def wait_idx(sv, dv, sem):
            pltpu.make_async_copy(src_hbm.at[cid, sid, 0], sv, sem).wait()
            pltpu.make_async_copy(dst_hbm.at[cid, sid, 0], dv, sem).wait()

        pltpu.sync_copy(src_hbm.at[cid, sid, 0], src_a)
        pltpu.sync_copy(dst_hbm.at[cid, sid, 0], dst_a)
        fetch_idx(1, src_b, dst_b, isem_b)
        pltpu.async_copy(table_hbm.at[src_a], rows_a, gsem_a)

        def body(t, carry):
            b0 = 2 * t
            pltpu.make_async_copy(table_hbm.at[src_a], rows_a, gsem_a).wait()
            wait_idx(src_b, dst_b, isem_b)
            pltpu.async_copy(table_hbm.at[src_b], rows_b, gsem_b)
            pltpu.sync_copy(rows_a, acc_f.at[dst_a], add=True)

            @pl.when(t < NB // 2 - 1)
            def _():
                fetch_idx(b0 + 2, src_a, dst_a, isem_a)
                wait_idx(src_a, dst_a, isem_a)
                pltpu.async_copy(table_hbm.at[src_a], rows_a, gsem_a)

            pltpu.make_async_copy(table_hbm.at[src_b], rows_b, gsem_b).wait()
            pltpu.sync_copy(rows_b, acc_f.at[dst_b], add=True)

            @pl.when(t < NB // 2 - 1)
            def _():
                fetch_idx(b0 + 3, src_b, dst_b, isem_b)

            return carry

        lax.fori_loop(0, NB // 2, body, 0)
        plsc.subcore_barrier()
        dump(feat_out)
        plsc.subcore_barrier()

        zero_acc()
        pltpu.sync_copy(ones_hbm, rows_a)
        plsc.subcore_barrier()

        pltpu.sync_copy(dst_hbm.at[cid, sid, 0], dst_a)
        pltpu.async_copy(dst_hbm.at[cid, sid, 1], dst_b, isem_b)

        def dbody(t, carry):
            b0 = 2 * t
            s_a = pltpu.async_copy(rows_a, acc_f.at[dst_a], ssem_a, add=True)
            pltpu.make_async_copy(dst_hbm.at[cid, sid, 0], dst_b, isem_b).wait()
            s_b = pltpu.async_copy(rows_a, acc_f.at[dst_b], ssem_b, add=True)
            s_a.wait()

            @pl.when(t < NB // 2 - 1)
            def _():
                pltpu.async_copy(dst_hbm.at[cid, sid, b0 + 2], dst_a, isem_a)
                pltpu.make_async_copy(dst_hbm.at[cid, sid, 0], dst_a,
                                      isem_a).wait()

            s_b.wait()

            @pl.when(t < NB // 2 - 1)
            def _():
                pltpu.async_copy(dst_hbm.at[cid, sid, b0 + 3], dst_b, isem_b)

            return carry

        lax.fori_loop(0, NB // 2, dbody, 0)
        plsc.subcore_barrier()
        dump(deg_out)

    return k(table, src_idx, dst_idx, zrows, ones_rows)


def _tc_epilogue_body(feat, dall, wc, ws, out):
    deg0 = jnp.maximum(dall[0, :, 0:1], 1.0)
    deg1 = jnp.maximum(dall[1, :, 0:1], 1.0)
    a0 = feat[0] / deg0
    a1 = feat[1] / deg1
    dn = (((1,), (0,)), ((), ()))
    p = lax.dot_general(a0, wc[...], dn, precision=lax.Precision.HIGHEST,
                        preferred_element_type=jnp.float32)
    q = lax.dot_general(a1, ws[...], dn, precision=lax.Precision.HIGHEST,
                        preferred_element_type=jnp.float32)
    out[...] = 0.5 * (p + q)


def _tc_epilogue(feat, deg, W_clicks, W_similar):
    BR = 2528
    grid = (ACC_ROWS // BR,)
    out = pl.pallas_call(
        _tc_epilogue_body,
        grid=grid,
        in_specs=[
            pl.BlockSpec((2, BR, D), lambda i: (0, i, 0)),
            pl.BlockSpec((2, BR, D), lambda i: (0, i, 0)),
            pl.BlockSpec((D, D), lambda i: (0, 0)),
            pl.BlockSpec((D, D), lambda i: (0, 0)),
        ],
        out_specs=pl.BlockSpec((BR, D), lambda i: (i, 0)),
        out_shape=jax.ShapeDtypeStruct((ACC_ROWS, D), jnp.float32),
    )(feat, deg, W_clicks, W_similar)
    return out[:N_ITEM]


def kernel(x_user, x_item, edge_index_clicks, edge_index_similar,
           W_clicks, W_similar):
    table = jnp.concatenate([x_user, x_item], axis=0)

    src_c = edge_index_clicks[0].astype(jnp.int32)
    dst_c = edge_index_clicks[1].astype(jnp.int32)
    src_s = edge_index_similar[0].astype(jnp.int32) + N_USER
    dst_s = edge_index_similar[1].astype(jnp.int32)

    pad = E_PAD - E
    if pad:
        src_c = jnp.concatenate([src_c, jnp.zeros((pad,), jnp.int32)])
        src_s = jnp.concatenate([src_s, jnp.full((pad,), N_USER, jnp.int32)])
        dpad = jnp.full((pad,), N_ITEM, jnp.int32)
        dst_c = jnp.concatenate([dst_c, dpad])
        dst_s = jnp.concatenate([dst_s, dpad])

    src_idx = jnp.stack([src_c, src_s]).reshape(2, NS, NB, BLK)
    dst_idx = jnp.stack([dst_c, dst_s]).reshape(2, NS, NB, BLK)

    zrows = jnp.zeros((16, D), jnp.float32)
    ones_rows = jnp.ones((BLK, D), jnp.float32)

    feat, deg = _sc_aggregate(table, src_idx, dst_idx, zrows, ones_rows)
    return _tc_epilogue(feat, deg, W_clicks, W_similar)


# speedup vs baseline: 1.8840x; 1.0104x over previous
import functools

import jax
import jax.numpy as jnp
from jax import lax
from jax.experimental import pallas as pl
from jax.experimental.pallas import tpu as pltpu
from jax.experimental.pallas import tpu_sc as plsc

D = 128
N_ITEM = 10000
N_USER = 10000
E = 320000

NS = 16
BLK = 176
NB = 114
E_PAD = NS * NB * BLK
ACC_ROWS = 10112
ROWS_PER_TILE = ACC_ROWS // NS


def _sc_aggregate(table, src_idx, dst_idx, zrows, ones_rows):
    mesh = plsc.VectorSubcoreMesh(core_axis_name="c", subcore_axis_name="s")

    @functools.partial(
        pl.kernel,
        out_type=(
            jax.ShapeDtypeStruct((2, ACC_ROWS, D), jnp.float32),
            jax.ShapeDtypeStruct((2, ACC_ROWS, D), jnp.float32),
        ),
        mesh=mesh,
        scratch_types=[
            pltpu.VMEM((BLK,), jnp.int32),
            pltpu.VMEM((BLK,), jnp.int32),
            pltpu.VMEM((BLK,), jnp.int32),
            pltpu.VMEM((BLK,), jnp.int32),
            pltpu.VMEM((BLK, D), jnp.float32),
            pltpu.VMEM((BLK, D), jnp.float32),
            pltpu.VMEM((16, D), jnp.float32),
            pltpu.VMEM_SHARED((ACC_ROWS, D), jnp.float32),
            pltpu.SemaphoreType.DMA,
            pltpu.SemaphoreType.DMA,
            pltpu.SemaphoreType.DMA,
            pltpu.SemaphoreType.DMA,
            pltpu.SemaphoreType.DMA,
            pltpu.SemaphoreType.DMA,
        ],
    )
    def k(table_hbm, src_hbm, dst_hbm, z_hbm, ones_hbm, feat_out, deg_out,
          src_a, src_b, dst_a, dst_b, rows_a, rows_b, zf_v, acc_f,
          isem_a, isem_b, gsem_a, gsem_b, ssem_a, ssem_b):
        cid = lax.axis_index("c")
        sid = lax.axis_index("s")

        pltpu.sync_copy(z_hbm, zf_v)

        def zero_acc():
            def zbody(t, carry):
                r = sid * ROWS_PER_TILE + t * 16
                pltpu.sync_copy(zf_v, acc_f.at[pl.ds(r, 16)])
                return carry

            lax.fori_loop(0, ROWS_PER_TILE // 16, zbody, 0)
            rem = ROWS_PER_TILE % 16
            if rem:
                r = sid * ROWS_PER_TILE + (ROWS_PER_TILE // 16) * 16
                pltpu.sync_copy(zf_v.at[pl.ds(0, rem)], acc_f.at[pl.ds(r, rem)])

        def dump(out_ref):
            r0 = sid * ROWS_PER_TILE
            pltpu.sync_copy(acc_f.at[pl.ds(r0, ROWS_PER_TILE)],
                            out_ref.at[cid, pl.ds(r0, ROWS_PER_TILE)])

        zero_acc()
        plsc.subcore_barrier()

        def fetch_idx(b, sv, dv, sem):
            s1 = pltpu.async_copy(src_hbm.at[cid, sid, b], sv, sem)
            s2 = pltpu.async_copy(dst_hbm.at[cid, sid, b], dv, sem)
            return s1, s2

        def wait_idx(sv, dv, sem):
            pltpu.make_async_copy(src_hbm.at[cid, sid, 0], sv, sem).wait()
            pltpu.make_async_copy(dst_hbm.at[cid, sid, 0], dv, sem).wait()

        pltpu.sync_copy(src_hbm.at[cid, sid, 0], src_a)
        pltpu.sync_copy(dst_hbm.at[cid, sid, 0], dst_a)
        fetch_idx(1, src_b, dst_b, isem_b)
        pltpu.async_copy(table_hbm.at[src_a], rows_a, gsem_a)

        def body(t, carry):
            b0 = 2 * t
            wait_idx(src_b, dst_b, isem_b)
            pltpu.async_copy(table_hbm.at[src_b], rows_b, gsem_b)
            pltpu.make_async_copy(table_hbm.at[src_a], rows_a, gsem_a).wait()
            pltpu.sync_copy(rows_a, acc_f.at[dst_a], add=True)

            @pl.when(t < NB // 2 - 1)
            def _():
                fetch_idx(b0 + 2, src_a, dst_a, isem_a)
                wait_idx(src_a, dst_a, isem_a)
                pltpu.async_copy(table_hbm.at[src_a], rows_a, gsem_a)

            pltpu.make_async_copy(table_hbm.at[src_b], rows_b, gsem_b).wait()
            pltpu.sync_copy(rows_b, acc_f.at[dst_b], add=True)

            @pl.when(t < NB // 2 - 1)
            def _():
                fetch_idx(b0 + 3, src_b, dst_b, isem_b)

            return carry

        lax.fori_loop(0, NB // 2, body, 0)
        plsc.subcore_barrier()
        dump(feat_out)
        plsc.subcore_barrier()

        zero_acc()
        pltpu.sync_copy(ones_hbm, rows_a)
        plsc.subcore_barrier()

        pltpu.sync_copy(dst_hbm.at[cid, sid, 0], dst_a)
        pltpu.async_copy(dst_hbm.at[cid, sid, 1], dst_b, isem_b)

        def dbody(t, carry):
            b0 = 2 * t
            s_a = pltpu.async_copy(rows_a, acc_f.at[dst_a], ssem_a, add=True)
            pltpu.make_async_copy(dst_hbm.at[cid, sid, 0], dst_b, isem_b).wait()
            s_b = pltpu.async_copy(rows_a, acc_f.at[dst_b], ssem_b, add=True)
            s_a.wait()

            @pl.when(t < NB // 2 - 1)
            def _():
                pltpu.async_copy(dst_hbm.at[cid, sid, b0 + 2], dst_a, isem_a)
                pltpu.make_async_copy(dst_hbm.at[cid, sid, 0], dst_a,
                                      isem_a).wait()

            s_b.wait()

            @pl.when(t < NB // 2 - 1)
            def _():
                pltpu.async_copy(dst_hbm.at[cid, sid, b0 + 3], dst_b, isem_b)

            return carry

        lax.fori_loop(0, NB // 2, dbody, 0)
        plsc.subcore_barrier()
        dump(deg_out)

    return k(table, src_idx, dst_idx, zrows, ones_rows)


def _tc_epilogue_body(feat, dall, wc, ws, out):
    deg0 = jnp.maximum(dall[0, :, 0:1], 1.0)
    deg1 = jnp.maximum(dall[1, :, 0:1], 1.0)
    a0 = feat[0] / deg0
    a1 = feat[1] / deg1
    dn = (((1,), (0,)), ((), ()))
    p = lax.dot_general(a0, wc[...], dn, precision=lax.Precision.HIGHEST,
                        preferred_element_type=jnp.float32)
    q = lax.dot_general(a1, ws[...], dn, precision=lax.Precision.HIGHEST,
                        preferred_element_type=jnp.float32)
    out[...] = 0.5 * (p + q)


def _tc_epilogue(feat, deg, W_clicks, W_similar):
    BR = 2528
    grid = (ACC_ROWS // BR,)
    out = pl.pallas_call(
        _tc_epilogue_body,
        grid=grid,
        in_specs=[
            pl.BlockSpec((2, BR, D), lambda i: (0, i, 0)),
            pl.BlockSpec((2, BR, D), lambda i: (0, i, 0)),
            pl.BlockSpec((D, D), lambda i: (0, 0)),
            pl.BlockSpec((D, D), lambda i: (0, 0)),
        ],
        out_specs=pl.BlockSpec((BR, D), lambda i: (i, 0)),
        out_shape=jax.ShapeDtypeStruct((ACC_ROWS, D), jnp.float32),
    )(feat, deg, W_clicks, W_similar)
    return out[:N_ITEM]


def kernel(x_user, x_item, edge_index_clicks, edge_index_similar,
           W_clicks, W_similar):
    table = jnp.concatenate([x_user, x_item], axis=0)

    src_c = edge_index_clicks[0].astype(jnp.int32)
    dst_c = edge_index_clicks[1].astype(jnp.int32)
    src_s = edge_index_similar[0].astype(jnp.int32) + N_USER
    dst_s = edge_index_similar[1].astype(jnp.int32)

    pad = E_PAD - E
    if pad:
        src_c = jnp.concatenate([src_c, jnp.zeros((pad,), jnp.int32)])
        src_s = jnp.concatenate([src_s, jnp.full((pad,), N_USER, jnp.int32)])
        dpad = jnp.full((pad,), N_ITEM, jnp.int32)
        dst_c = jnp.concatenate([dst_c, dpad])
        dst_s = jnp.concatenate([dst_s, dpad])

    src_idx = jnp.stack([src_c, src_s]).reshape(2, NS, NB, BLK)
    dst_idx = jnp.stack([dst_c, dst_s]).reshape(2, NS, NB, BLK)

    zrows = jnp.zeros((16, D), jnp.float32)
    ones_rows = jnp.ones((BLK, D), jnp.float32)

    feat, deg = _sc_aggregate(table, src_idx, dst_idx, zrows, ones_rows)
    return _tc_epilogue(feat, deg, W_clicks, W_similar)
